# Initial kernel scaffold; baseline (speedup 1.0000x reference)
import jax
import jax.numpy as jnp
from jax.experimental import pallas as pl


def kernel(x, edge_index, batch, W1, a_src1, a_dst1, b1, W2, a_src2, a_dst2, b2, fc1_w, fc1_b, fc2_w, fc2_b):
    raise NotImplementedError("write your pallas kernel here")


# speedup vs baseline: 9.1495x; 9.1495x over previous
import functools

import jax
import jax.numpy as jnp
from jax import lax
from jax.experimental import pallas as pl
from jax.experimental.pallas import tpu as pltpu
from jax.experimental.pallas import tpu_sc as plsc


def _proj_kernel(x_ref, w_ref, a_ref, h_ref, al_ref):
    j = pl.program_id(1)
    hblk = jnp.dot(x_ref[...], w_ref[...], preferred_element_type=jnp.float32,
                   precision=lax.Precision.HIGHEST)
    h_ref[0] = hblk[:, :64]
    h_ref[1] = hblk[:, 64:]

    @pl.when(j == 0)
    def _():
        al_ref[...] = jnp.zeros_like(al_ref)

    al_ref[...] += jnp.dot(hblk, a_ref[...], preferred_element_type=jnp.float32,
                           precision=lax.Precision.HIGHEST)


def _run_proj(xp, W, Ap, RB):
    NP, K = xp.shape
    C = W.shape[1] // 128
    grid = (NP // RB, C)
    return pl.pallas_call(
        _proj_kernel,
        grid=grid,
        in_specs=[
            pl.BlockSpec((RB, K), lambda i, j: (i, 0)),
            pl.BlockSpec((K, 128), lambda i, j: (0, j)),
            pl.BlockSpec((128, 128), lambda i, j: (j, 0)),
        ],
        out_specs=[
            pl.BlockSpec((2, RB, 64), lambda i, j: (j, i, 0)),
            pl.BlockSpec((RB, 128), lambda i, j: (i, 0)),
        ],
        out_shape=[
            jax.ShapeDtypeStruct((2 * C, NP, 64), jnp.float32),
            jax.ShapeDtypeStruct((NP, 128), jnp.float32),
        ],
    )(xp, W, Ap)


def _proj2_kernel(o_ref, b_ref, w_ref, a_ref, h_ref, al_ref):
    j = pl.program_id(1)
    hv = jnp.maximum(o_ref[...] + b_ref[...][:, None, :], 0.0)
    acc = jnp.zeros((hv.shape[1], 128), jnp.float32)
    for cc in range(8):
        acc += jnp.dot(hv[cc], w_ref[cc], preferred_element_type=jnp.float32,
                       precision=lax.Precision.HIGHEST)
    h_ref[0] = acc[:, :64]
    h_ref[1] = acc[:, 64:]

    @pl.when(j == 0)
    def _():
        al_ref[...] = jnp.zeros_like(al_ref)

    al_ref[...] += jnp.dot(acc, a_ref[...], preferred_element_type=jnp.float32,
                           precision=lax.Precision.HIGHEST)


def _run_proj2(o1, b1r, W2r, A2r, RB):
    NP = o1.shape[1]
    C = W2r.shape[2] // 128
    grid = (NP // RB, C)
    return pl.pallas_call(
        _proj2_kernel,
        grid=grid,
        in_specs=[
            pl.BlockSpec((8, RB, 64), lambda i, j: (0, i, 0)),
            pl.BlockSpec((8, 64), lambda i, j: (0, 0)),
            pl.BlockSpec((8, 64, 128), lambda i, j: (0, 0, j)),
            pl.BlockSpec((128, 128), lambda i, j: (j, 0)),
        ],
        out_specs=[
            pl.BlockSpec((2, RB, 64), lambda i, j: (j, i, 0)),
            pl.BlockSpec((RB, 128), lambda i, j: (i, 0)),
        ],
        out_shape=[
            jax.ShapeDtypeStruct((2 * C, NP, 64), jnp.float32),
            jax.ShapeDtypeStruct((NP, 128), jnp.float32),
        ],
    )(o1, b1r, W2r, A2r)


def _pool_kernel(o_ref, b_ref, bat_ref, f1w_ref, f1b_ref, f2w_ref, f2b_ref,
                 out_ref, acc_ref, cnt_ref):
    i = pl.program_id(0)
    nblocks = pl.num_programs(0)

    @pl.when(i == 0)
    def _():
        acc_ref[...] = jnp.zeros_like(acc_ref)
        cnt_ref[...] = jnp.zeros_like(cnt_ref)

    bvec = bat_ref[0, 0, :]
    validc = bvec[:, None] < 32
    oh = (bvec[:, None] == lax.broadcasted_iota(jnp.int32, (bvec.shape[0], 32), 1))
    oh = oh.astype(jnp.float32)
    for cc in range(8):
        hv = jnp.maximum(o_ref[cc] + b_ref[...][cc][None, :], 0.0)
        hv = jnp.where(validc, hv, 0.0)
        acc_ref[:, cc * 64:(cc + 1) * 64] += lax.dot_general(
            oh, hv, (((0,), (0,)), ((), ())),
            preferred_element_type=jnp.float32,
            precision=lax.Precision.HIGHEST)
    cnt_ref[0, :32] += jnp.sum(oh, axis=0)

    @pl.when(i == nblocks - 1)
    def _():
        cnt = jnp.maximum(cnt_ref[0, :32], 1.0)
        pooled = acc_ref[...] / cnt[:, None]
        g = jnp.maximum(
            jnp.dot(pooled, f1w_ref[...], preferred_element_type=jnp.float32,
                    precision=lax.Precision.HIGHEST) + f1b_ref[...], 0.0)
        out_ref[...] = jnp.dot(
            g, f2w_ref[...], preferred_element_type=jnp.float32,
            precision=lax.Precision.HIGHEST) + f2b_ref[...]


def _run_pool(o2, b2r, bat3, fc1_w, fc1_b, fc2_w, fc2_b, RB):
    NP = o2.shape[1]
    grid = (NP // RB,)
    return pl.pallas_call(
        _pool_kernel,
        grid=grid,
        in_specs=[
            pl.BlockSpec((8, RB, 64), lambda i: (0, i, 0)),
            pl.BlockSpec((8, 64), lambda i: (0, 0)),
            pl.BlockSpec((1, 1, RB), lambda i: (i, 0, 0)),
            pl.BlockSpec(fc1_w.shape, lambda i: (0, 0)),
            pl.BlockSpec((1, 1024), lambda i: (0, 0)),
            pl.BlockSpec(fc2_w.shape, lambda i: (0, 0)),
            pl.BlockSpec((1, 128), lambda i: (0, 0)),
        ],
        out_specs=pl.BlockSpec((32, 128), lambda i: (0, 0)),
        out_shape=jax.ShapeDtypeStruct((32, 128), jnp.float32),
        scratch_shapes=[
            pltpu.VMEM((32, 512), jnp.float32),
            pltpu.VMEM((8, 128), jnp.float32),
        ],
    )(o2, b2r, bat3, fc1_w, fc1_b, fc2_w, fc2_b)


def _make_sc_layer(H, NP, EP):
    NBT = EP // 128 // 16
    NBW = NBT
    STRIP = 8
    RPT = NP // 16
    DEN = NP * H
    mesh = plsc.VectorSubcoreMesh(core_axis_name="c", subcore_axis_name="s")

    def body(src3_hbm, dst3_hbm, asrc_hbm, adst_hbm, h_hbm, out_hbm,
             src3_v, dst3_v, asrc_v, adst_v, den_v, alpha_v,
             vals_v, idx_v, rbuf, zbuf, zed, acc_sh, den_sh, semg, sema):
        c = lax.axis_index("c")
        s = lax.axis_index("s")

        pltpu.sync_copy(src3_hbm.at[pl.ds(s * NBT, NBT)], src3_v)
        pltpu.sync_copy(dst3_hbm.at[pl.ds(s * NBT, NBT)], dst3_v)

        def _z16(i, _):
            zed[pl.ds(i * 16, 16)] = jnp.zeros((16,), jnp.float32)
            return 0
        lax.fori_loop(0, zed.shape[0] // 16, _z16, 0)

        def _zb(i, _):
            for k in range(zbuf.shape[1] // 16):
                zbuf[i, pl.ds(k * 16, 16)] = jnp.zeros((16,), jnp.float32)
            return 0
        lax.fori_loop(0, zbuf.shape[0], _zb, 0)

        seg = DEN // 16
        pltpu.sync_copy(zed.at[pl.ds(0, seg)], den_sh.at[pl.ds(s * seg, seg)])
        plsc.subcore_barrier()

        for hh in range(H):
            pltpu.sync_copy(asrc_hbm.at[hh], asrc_v)
            pltpu.sync_copy(adst_hbm.at[hh], adst_v)

            def strip_body(t, _):
                descs = []
                for b in range(STRIP):
                    bat = t * STRIP + b
                    for u in range(8):
                        sv = src3_v[bat, 0, pl.ds(u * 16, 16)]
                        dv = dst3_v[bat, 0, pl.ds(u * 16, 16)]
                        a_s = plsc.load_gather(asrc_v, [sv])
                        a_d = plsc.load_gather(adst_v, [dv])
                        e = a_s + a_d
                        e = jnp.where(e >= 0.0, e, 0.2 * e)
                        ex = jnp.exp(e)
                        vals_v[b, 0, pl.ds(u * 16, 16)] = ex
                        idx_v[b, 0, pl.ds(u * 16, 16)] = dv + hh * NP
                for b in range(STRIP):
                    descs.append(pltpu.async_copy(
                        vals_v.at[b, 0], den_sh.at[idx_v.at[b, 0]], sema,
                        add=True))
                for d in descs:
                    d.wait()
                return 0
            lax.fori_loop(0, NBT // STRIP, strip_body, 0)
        plsc.subcore_barrier()

        hd = c if H == 2 else c * 0
        if H == 2:
            pltpu.sync_copy(asrc_hbm.at[hd], asrc_v)
            pltpu.sync_copy(adst_hbm.at[hd], adst_v)
        pltpu.sync_copy(den_sh.at[pl.ds(hd * NP, NP)], den_v)

        def alpha_body(st, _):
            bi = st // 8
            lo = (st % 8) * 16
            sv = src3_v[bi, 0, pl.ds(lo, 16)]
            dv = dst3_v[bi, 0, pl.ds(lo, 16)]
            a_s = plsc.load_gather(asrc_v, [sv])
            a_d = plsc.load_gather(adst_v, [dv])
            e = a_s + a_d
            e = jnp.where(e >= 0.0, e, 0.2 * e)
            ex = jnp.exp(e)
            den = plsc.load_gather(den_v, [dv])
            alpha_v[pl.ds(st * 16, 16)] = ex / jnp.maximum(den, 1e-16)
            return 0
        lax.fori_loop(0, NBW * 8, alpha_body, 0)

        for half in range(4):
            q = 4 * c + half
            for k in range(RPT // 64):
                pltpu.sync_copy(zbuf, acc_sh.at[pl.ds(s * RPT + k * 64, 64)])
            plsc.subcore_barrier()

            def msg_body(b, _):
                @pl.when(b > 0)
                def _():
                    pb = b - 1
                    pltpu.make_async_copy(
                        h_hbm.at[q].at[src3_v.at[pb, 0]],
                        rbuf.at[lax.rem(pb, 2)], semg).wait()

                    def scale_grp(g, _):
                        av16 = alpha_v[pl.ds(pb * 128 + g * 16, 16)]
                        for j in range(16):
                            av = av16[j]
                            for k in range(4):
                                rbuf[lax.rem(pb, 2), g * 16 + j,
                                     pl.ds(k * 16, 16)] = (
                                    rbuf[lax.rem(pb, 2), g * 16 + j,
                                         pl.ds(k * 16, 16)] * av)
                        return 0
                    lax.fori_loop(0, 8, scale_grp, 0)
                    pltpu.async_copy(
                        rbuf.at[lax.rem(pb, 2)],
                        acc_sh.at[dst3_v.at[pb, 0]], sema,
                        add=True).wait()

                @pl.when(b < NBW)
                def _():
                    pltpu.async_copy(
                        h_hbm.at[q].at[src3_v.at[b, 0]],
                        rbuf.at[lax.rem(b, 2)], semg)
                return 0
            lax.fori_loop(0, NBW + 1, msg_body, 0)
            plsc.subcore_barrier()

            for k in range(RPT // 128):
                r0 = s * RPT + k * 128
                pltpu.sync_copy(acc_sh.at[pl.ds(r0, 128)], rbuf.at[0])
                pltpu.sync_copy(rbuf.at[0], out_hbm.at[q].at[pl.ds(r0, 128)])
            plsc.subcore_barrier()

    kern = pl.kernel(
        body,
        out_type=jax.ShapeDtypeStruct((8, NP, 64), jnp.float32),
        mesh=mesh,
        compiler_params=pltpu.CompilerParams(needs_layout_passes=False,
                                             use_tc_tiling_on_sc=False),
        scratch_types=[
            pltpu.VMEM((NBT, 1, 128), jnp.int32),
            pltpu.VMEM((NBT, 1, 128), jnp.int32),
            pltpu.VMEM((NP,), jnp.float32),
            pltpu.VMEM((NP,), jnp.float32),
            pltpu.VMEM((NP,), jnp.float32),
            pltpu.VMEM((NBW * 128,), jnp.float32),
            pltpu.VMEM((STRIP, 1, 128), jnp.float32),
            pltpu.VMEM((STRIP, 1, 128), jnp.int32),
            pltpu.VMEM((2, 128, 64), jnp.float32),
            pltpu.VMEM((64, 64), jnp.float32),
            pltpu.VMEM((DEN // 16,), jnp.float32),
            pltpu.VMEM_SHARED((NP, 64), jnp.float32),
            pltpu.VMEM_SHARED((DEN,), jnp.float32),
            pltpu.SemaphoreType.DMA,
            pltpu.SemaphoreType.DMA,
        ],
    )
    return kern


def kernel(x, edge_index, batch, W1, a_src1, a_dst1, b1,
           W2, a_src2, a_dst2, b2, fc1_w, fc1_b, fc2_w, fc2_b):
    N, F_in = x.shape
    E = edge_index.shape[1]
    H1, D1 = a_src1.shape
    D2 = W2.shape[1] // 2
    G = 32
    RB = 512

    NP = ((N + 8 + 2047) // 2048) * 2048
    EP = ((E + 4095) // 4096) * 4096

    f32 = jnp.float32
    src = edge_index[0].astype(jnp.int32)
    dst = edge_index[1].astype(jnp.int32)
    pad = EP - E
    ii = jnp.arange(pad, dtype=jnp.int32)
    src_p = jnp.concatenate([src, (ii * 37) % N])
    dst_p = jnp.concatenate([dst, N + (ii % 8)])
    src3 = src_p.reshape(EP // 128, 1, 128)
    dst3 = dst_p.reshape(EP // 128, 1, 128)

    xp = jnp.pad(x.astype(f32), ((0, NP - N), (0, 0)))

    A1p = jnp.zeros((H1 * D1, 128), f32)
    for h in range(H1):
        A1p = A1p.at[h * D1:(h + 1) * D1, h].set(a_src1[h])
        A1p = A1p.at[h * D1:(h + 1) * D1, 2 + h].set(a_dst1[h])

    h1tab, al1 = _run_proj(xp, W1.astype(f32), A1p, RB)
    asrc1_t = al1[:, 0:2].T.reshape(2, NP)
    adst1_t = al1[:, 2:4].T.reshape(2, NP)

    sc1 = _make_sc_layer(2, NP, EP)
    o1tab = sc1(src3, dst3, asrc1_t, adst1_t, h1tab)

    b1r = b1.astype(f32).reshape(8, 64)
    W2r = W2.astype(f32).reshape(8, 64, W2.shape[1])
    A2p = jnp.zeros((2 * D2, 128), f32)
    A2p = A2p.at[:, 0].set(a_src2[0])
    A2p = A2p.at[:, 1].set(a_dst2[0])
    h2tab, al2 = _run_proj2(o1tab, b1r, W2r, A2p, RB)
    asrc2_t = al2[:, 0].reshape(1, NP)
    adst2_t = al2[:, 1].reshape(1, NP)

    sc2 = _make_sc_layer(1, NP, EP)
    o2tab = sc2(src3, dst3, asrc2_t, adst2_t, h2tab)

    b2r = b2.astype(f32).reshape(8, 64)
    batp = jnp.pad(batch.astype(jnp.int32), (0, NP - N), constant_values=G)
    bat3 = batp.reshape(NP // RB, 1, RB)

    out = _run_pool(o2tab, b2r, bat3,
                    fc1_w.astype(f32), fc1_b.astype(f32).reshape(1, 1024),
                    fc2_w.astype(f32), fc2_b.astype(f32).reshape(1, 128), RB)
    return out


# speedup vs baseline: 9.9012x; 1.0822x over previous
import functools

import jax
import jax.numpy as jnp
from jax import lax
from jax.experimental import pallas as pl
from jax.experimental.pallas import tpu as pltpu
from jax.experimental.pallas import tpu_sc as plsc


def _proj_kernel(x_ref, w_ref, a_ref, h_ref, al_ref):
    j = pl.program_id(1)
    hblk = jnp.dot(x_ref[...], w_ref[...], preferred_element_type=jnp.float32,
                   precision=lax.Precision.HIGHEST)
    h_ref[0] = hblk[:, :64]
    h_ref[1] = hblk[:, 64:]

    @pl.when(j == 0)
    def _():
        al_ref[...] = jnp.zeros_like(al_ref)

    al_ref[...] += jnp.dot(hblk, a_ref[...], preferred_element_type=jnp.float32,
                           precision=lax.Precision.HIGHEST)


def _run_proj(xp, W, Ap, RB):
    NP, K = xp.shape
    C = W.shape[1] // 128
    grid = (NP // RB, C)
    return pl.pallas_call(
        _proj_kernel,
        grid=grid,
        in_specs=[
            pl.BlockSpec((RB, K), lambda i, j: (i, 0)),
            pl.BlockSpec((K, 128), lambda i, j: (0, j)),
            pl.BlockSpec((128, 128), lambda i, j: (j, 0)),
        ],
        out_specs=[
            pl.BlockSpec((2, RB, 64), lambda i, j: (j, i, 0)),
            pl.BlockSpec((RB, 128), lambda i, j: (i, 0)),
        ],
        out_shape=[
            jax.ShapeDtypeStruct((2 * C, NP, 64), jnp.float32),
            jax.ShapeDtypeStruct((NP, 128), jnp.float32),
        ],
    )(xp, W, Ap)


def _proj2_kernel(o_ref, b_ref, w_ref, a_ref, h_ref, al_ref):
    j = pl.program_id(1)
    hv = jnp.maximum(o_ref[...] + b_ref[...][:, None, :], 0.0)
    acc = jnp.zeros((hv.shape[1], 128), jnp.float32)
    for cc in range(8):
        acc += jnp.dot(hv[cc], w_ref[cc], preferred_element_type=jnp.float32,
                       precision=lax.Precision.HIGHEST)
    h_ref[0] = acc[:, :64]
    h_ref[1] = acc[:, 64:]

    @pl.when(j == 0)
    def _():
        al_ref[...] = jnp.zeros_like(al_ref)

    al_ref[...] += jnp.dot(acc, a_ref[...], preferred_element_type=jnp.float32,
                           precision=lax.Precision.HIGHEST)


def _run_proj2(o1, b1r, W2r, A2r, RB):
    NP = o1.shape[1]
    C = W2r.shape[2] // 128
    grid = (NP // RB, C)
    return pl.pallas_call(
        _proj2_kernel,
        grid=grid,
        in_specs=[
            pl.BlockSpec((8, RB, 64), lambda i, j: (0, i, 0)),
            pl.BlockSpec((8, 64), lambda i, j: (0, 0)),
            pl.BlockSpec((8, 64, 128), lambda i, j: (0, 0, j)),
            pl.BlockSpec((128, 128), lambda i, j: (j, 0)),
        ],
        out_specs=[
            pl.BlockSpec((2, RB, 64), lambda i, j: (j, i, 0)),
            pl.BlockSpec((RB, 128), lambda i, j: (i, 0)),
        ],
        out_shape=[
            jax.ShapeDtypeStruct((2 * C, NP, 64), jnp.float32),
            jax.ShapeDtypeStruct((NP, 128), jnp.float32),
        ],
    )(o1, b1r, W2r, A2r)


def _pool_kernel(o_ref, b_ref, bat_ref, f1w_ref, f1b_ref, f2w_ref, f2b_ref,
                 out_ref, acc_ref, cnt_ref):
    i = pl.program_id(0)
    nblocks = pl.num_programs(0)

    @pl.when(i == 0)
    def _():
        acc_ref[...] = jnp.zeros_like(acc_ref)
        cnt_ref[...] = jnp.zeros_like(cnt_ref)

    bvec = bat_ref[0, 0, :]
    validc = bvec[:, None] < 32
    oh = (bvec[:, None] == lax.broadcasted_iota(jnp.int32, (bvec.shape[0], 32), 1))
    oh = oh.astype(jnp.float32)
    for cc in range(8):
        hv = jnp.maximum(o_ref[cc] + b_ref[...][cc][None, :], 0.0)
        hv = jnp.where(validc, hv, 0.0)
        acc_ref[:, cc * 64:(cc + 1) * 64] += lax.dot_general(
            oh, hv, (((0,), (0,)), ((), ())),
            preferred_element_type=jnp.float32,
            precision=lax.Precision.HIGHEST)
    cnt_ref[0, :32] += jnp.sum(oh, axis=0)

    @pl.when(i == nblocks - 1)
    def _():
        cnt = jnp.maximum(cnt_ref[0, :32], 1.0)
        pooled = acc_ref[...] / cnt[:, None]
        g = jnp.maximum(
            jnp.dot(pooled, f1w_ref[...], preferred_element_type=jnp.float32,
                    precision=lax.Precision.HIGHEST) + f1b_ref[...], 0.0)
        out_ref[...] = jnp.dot(
            g, f2w_ref[...], preferred_element_type=jnp.float32,
            precision=lax.Precision.HIGHEST) + f2b_ref[...]


def _run_pool(o2, b2r, bat3, fc1_w, fc1_b, fc2_w, fc2_b, RB):
    NP = o2.shape[1]
    grid = (NP // RB,)
    return pl.pallas_call(
        _pool_kernel,
        grid=grid,
        in_specs=[
            pl.BlockSpec((8, RB, 64), lambda i: (0, i, 0)),
            pl.BlockSpec((8, 64), lambda i: (0, 0)),
            pl.BlockSpec((1, 1, RB), lambda i: (i, 0, 0)),
            pl.BlockSpec(fc1_w.shape, lambda i: (0, 0)),
            pl.BlockSpec((1, 1024), lambda i: (0, 0)),
            pl.BlockSpec(fc2_w.shape, lambda i: (0, 0)),
            pl.BlockSpec((1, 128), lambda i: (0, 0)),
        ],
        out_specs=pl.BlockSpec((32, 128), lambda i: (0, 0)),
        out_shape=jax.ShapeDtypeStruct((32, 128), jnp.float32),
        scratch_shapes=[
            pltpu.VMEM((32, 512), jnp.float32),
            pltpu.VMEM((8, 128), jnp.float32),
        ],
    )(o2, b2r, bat3, fc1_w, fc1_b, fc2_w, fc2_b)


def _make_sc_layer(H, NP, EP):
    NBT = EP // 128 // 16
    NBW = NBT
    STRIP = 8
    RSL = 2
    RPT = NP // 16
    DEN = NP * H
    mesh = plsc.VectorSubcoreMesh(core_axis_name="c", subcore_axis_name="s")

    def body(src3_hbm, dst3_hbm, asrc_hbm, adst_hbm, h_hbm, out_hbm,
             src3_v, dst3_v, asrc_v, adst_v, den_v, alpha_v,
             vals_v, idx_v, rbuf, zbuf, zed, acc_sh, den_sh, semg, sema):
        c = lax.axis_index("c")
        s = lax.axis_index("s")

        pltpu.sync_copy(src3_hbm.at[pl.ds(s * NBT, NBT)], src3_v)
        pltpu.sync_copy(dst3_hbm.at[pl.ds(s * NBT, NBT)], dst3_v)

        def _z16(i, _):
            zed[pl.ds(i * 16, 16)] = jnp.zeros((16,), jnp.float32)
            return 0
        lax.fori_loop(0, zed.shape[0] // 16, _z16, 0)

        def _zb(i, _):
            for k in range(zbuf.shape[1] // 16):
                zbuf[i, pl.ds(k * 16, 16)] = jnp.zeros((16,), jnp.float32)
            return 0
        lax.fori_loop(0, zbuf.shape[0], _zb, 0)

        seg = DEN // 16
        pltpu.sync_copy(zed.at[pl.ds(0, seg)], den_sh.at[pl.ds(s * seg, seg)])
        plsc.subcore_barrier()

        for hh in range(H):
            pltpu.sync_copy(asrc_hbm.at[hh], asrc_v)
            pltpu.sync_copy(adst_hbm.at[hh], adst_v)

            def strip_body(t, _):
                descs = []
                for b in range(STRIP):
                    bat = t * STRIP + b
                    for u in range(8):
                        sv = src3_v[bat, 0, pl.ds(u * 16, 16)]
                        dv = dst3_v[bat, 0, pl.ds(u * 16, 16)]
                        a_s = plsc.load_gather(asrc_v, [sv])
                        a_d = plsc.load_gather(adst_v, [dv])
                        e = a_s + a_d
                        e = jnp.where(e >= 0.0, e, 0.2 * e)
                        ex = jnp.exp(e)
                        vals_v[b, 0, pl.ds(u * 16, 16)] = ex
                        idx_v[b, 0, pl.ds(u * 16, 16)] = dv + hh * NP
                for b in range(STRIP):
                    descs.append(pltpu.async_copy(
                        vals_v.at[b, 0], den_sh.at[idx_v.at[b, 0]], sema,
                        add=True))
                for d in descs:
                    d.wait()
                return 0
            lax.fori_loop(0, NBT // STRIP, strip_body, 0)
        plsc.subcore_barrier()

        hd = c if H == 2 else c * 0
        if H == 2:
            pltpu.sync_copy(asrc_hbm.at[hd], asrc_v)
            pltpu.sync_copy(adst_hbm.at[hd], adst_v)
        pltpu.sync_copy(den_sh.at[pl.ds(hd * NP, NP)], den_v)

        def alpha_body(st, _):
            bi = st // 8
            lo = (st % 8) * 16
            sv = src3_v[bi, 0, pl.ds(lo, 16)]
            dv = dst3_v[bi, 0, pl.ds(lo, 16)]
            a_s = plsc.load_gather(asrc_v, [sv])
            a_d = plsc.load_gather(adst_v, [dv])
            e = a_s + a_d
            e = jnp.where(e >= 0.0, e, 0.2 * e)
            ex = jnp.exp(e)
            den = plsc.load_gather(den_v, [dv])
            alpha_v[pl.ds(st * 16, 16)] = ex / jnp.maximum(den, 1e-16)
            return 0
        lax.fori_loop(0, NBW * 8, alpha_body, 0)

        for half in range(4):
            q = 4 * c + half
            for k in range(RPT // 64):
                pltpu.sync_copy(zbuf, acc_sh.at[pl.ds(s * RPT + k * 64, 64)])
            plsc.subcore_barrier()

            def msg_body(b, _):
                @pl.when(b > 0)
                def _():
                    pb = b - 1
                    sl = lax.rem(pb, RSL)
                    pltpu.make_async_copy(
                        h_hbm.at[q].at[src3_v.at[pb, 0]],
                        rbuf.at[sl], semg).wait()

                    def scale_grp(g, _):
                        av16 = alpha_v[pl.ds(pb * 128 + g * 16, 16)]
                        for j in range(16):
                            av = av16[j]
                            for k in range(4):
                                rbuf[sl, g * 16 + j, pl.ds(k * 16, 16)] = (
                                    rbuf[sl, g * 16 + j,
                                         pl.ds(k * 16, 16)] * av)
                        return 0
                    lax.fori_loop(0, 8, scale_grp, 0)
                    pltpu.async_copy(
                        rbuf.at[sl], acc_sh.at[dst3_v.at[pb, 0]], sema,
                        add=True)

                @pl.when(b < NBW)
                def _():
                    @pl.when(b >= RSL)
                    def _():
                        ob = b - RSL
                        pltpu.make_async_copy(
                            rbuf.at[lax.rem(ob, RSL)],
                            acc_sh.at[dst3_v.at[ob, 0]], sema).wait()
                    pltpu.async_copy(
                        h_hbm.at[q].at[src3_v.at[b, 0]],
                        rbuf.at[lax.rem(b, RSL)], semg)
                return 0
            lax.fori_loop(0, NBW + 1, msg_body, 0)
            for t in range(RSL):
                ob = NBW - RSL + t
                pltpu.make_async_copy(
                    rbuf.at[ob % RSL], acc_sh.at[dst3_v.at[ob, 0]],
                    sema).wait()
            plsc.subcore_barrier()

            for k in range(RPT // 128):
                r0 = s * RPT + k * 128
                pltpu.sync_copy(acc_sh.at[pl.ds(r0, 128)], rbuf.at[0])
                pltpu.sync_copy(rbuf.at[0], out_hbm.at[q].at[pl.ds(r0, 128)])
            plsc.subcore_barrier()

    kern = pl.kernel(
        body,
        out_type=jax.ShapeDtypeStruct((8, NP, 64), jnp.float32),
        mesh=mesh,
        compiler_params=pltpu.CompilerParams(needs_layout_passes=False,
                                             use_tc_tiling_on_sc=False),
        scratch_types=[
            pltpu.VMEM((NBT, 1, 128), jnp.int32),
            pltpu.VMEM((NBT, 1, 128), jnp.int32),
            pltpu.VMEM((NP,), jnp.float32),
            pltpu.VMEM((NP,), jnp.float32),
            pltpu.VMEM((NP,), jnp.float32),
            pltpu.VMEM((NBW * 128,), jnp.float32),
            pltpu.VMEM((STRIP, 1, 128), jnp.float32),
            pltpu.VMEM((STRIP, 1, 128), jnp.int32),
            pltpu.VMEM((RSL, 128, 64), jnp.float32),
            pltpu.VMEM((64, 64), jnp.float32),
            pltpu.VMEM((DEN // 16,), jnp.float32),
            pltpu.VMEM_SHARED((NP, 64), jnp.float32),
            pltpu.VMEM_SHARED((DEN,), jnp.float32),
            pltpu.SemaphoreType.DMA,
            pltpu.SemaphoreType.DMA,
        ],
    )
    return kern


def kernel(x, edge_index, batch, W1, a_src1, a_dst1, b1,
           W2, a_src2, a_dst2, b2, fc1_w, fc1_b, fc2_w, fc2_b):
    N, F_in = x.shape
    E = edge_index.shape[1]
    H1, D1 = a_src1.shape
    D2 = W2.shape[1] // 2
    G = 32
    RB = 512

    NP = ((N + 8 + 2047) // 2048) * 2048
    EP = ((E + 4095) // 4096) * 4096

    f32 = jnp.float32
    src = edge_index[0].astype(jnp.int32)
    dst = edge_index[1].astype(jnp.int32)
    pad = EP - E
    ii = jnp.arange(pad, dtype=jnp.int32)
    src_p = jnp.concatenate([src, (ii * 37) % N])
    dst_p = jnp.concatenate([dst, N + (ii % 8)])
    src3 = src_p.reshape(EP // 128, 1, 128)
    dst3 = dst_p.reshape(EP // 128, 1, 128)

    xp = jnp.pad(x.astype(f32), ((0, NP - N), (0, 0)))

    A1p = jnp.zeros((H1 * D1, 128), f32)
    for h in range(H1):
        A1p = A1p.at[h * D1:(h + 1) * D1, h].set(a_src1[h])
        A1p = A1p.at[h * D1:(h + 1) * D1, 2 + h].set(a_dst1[h])

    h1tab, al1 = _run_proj(xp, W1.astype(f32), A1p, RB)
    asrc1_t = al1[:, 0:2].T.reshape(2, NP)
    adst1_t = al1[:, 2:4].T.reshape(2, NP)

    sc1 = _make_sc_layer(2, NP, EP)
    o1tab = sc1(src3, dst3, asrc1_t, adst1_t, h1tab)

    b1r = b1.astype(f32).reshape(8, 64)
    W2r = W2.astype(f32).reshape(8, 64, W2.shape[1])
    A2p = jnp.zeros((2 * D2, 128), f32)
    A2p = A2p.at[:, 0].set(a_src2[0])
    A2p = A2p.at[:, 1].set(a_dst2[0])
    h2tab, al2 = _run_proj2(o1tab, b1r, W2r, A2p, RB)
    asrc2_t = al2[:, 0].reshape(1, NP)
    adst2_t = al2[:, 1].reshape(1, NP)

    sc2 = _make_sc_layer(1, NP, EP)
    o2tab = sc2(src3, dst3, asrc2_t, adst2_t, h2tab)

    b2r = b2.astype(f32).reshape(8, 64)
    batp = jnp.pad(batch.astype(jnp.int32), (0, NP - N), constant_values=G)
    bat3 = batp.reshape(NP // RB, 1, RB)

    out = _run_pool(o2tab, b2r, bat3,
                    fc1_w.astype(f32), fc1_b.astype(f32).reshape(1, 1024),
                    fc2_w.astype(f32), fc2_b.astype(f32).reshape(1, 128), RB)
    return out


# speedup vs baseline: 10.4950x; 1.0600x over previous
import functools

import jax
import jax.numpy as jnp
from jax import lax
from jax.experimental import pallas as pl
from jax.experimental.pallas import tpu as pltpu
from jax.experimental.pallas import tpu_sc as plsc


def _proj_kernel(x_ref, w_ref, a_ref, h_ref, al_ref):
    j = pl.program_id(1)
    hblk = jnp.dot(x_ref[...], w_ref[...], preferred_element_type=jnp.float32)
    h_ref[0] = hblk[:, :64]
    h_ref[1] = hblk[:, 64:]

    @pl.when(j == 0)
    def _():
        al_ref[...] = jnp.zeros_like(al_ref)

    al_ref[...] += jnp.dot(hblk, a_ref[...], preferred_element_type=jnp.float32,
                           precision=lax.Precision.HIGHEST)


def _run_proj(xp, W, Ap, RB):
    NP, K = xp.shape
    C = W.shape[1] // 128
    grid = (NP // RB, C)
    return pl.pallas_call(
        _proj_kernel,
        grid=grid,
        in_specs=[
            pl.BlockSpec((RB, K), lambda i, j: (i, 0)),
            pl.BlockSpec((K, 128), lambda i, j: (0, j)),
            pl.BlockSpec((128, 128), lambda i, j: (j, 0)),
        ],
        out_specs=[
            pl.BlockSpec((2, RB, 64), lambda i, j: (j, i, 0)),
            pl.BlockSpec((RB, 128), lambda i, j: (i, 0)),
        ],
        out_shape=[
            jax.ShapeDtypeStruct((2 * C, NP, 64), jnp.float32),
            jax.ShapeDtypeStruct((NP, 128), jnp.float32),
        ],
    )(xp, W, Ap)


def _proj2_kernel(o_ref, b_ref, w_ref, a_ref, h_ref, al_ref):
    j = pl.program_id(1)
    hv = jnp.maximum(o_ref[...] + b_ref[...][:, None, :], 0.0)
    acc = jnp.zeros((hv.shape[1], 128), jnp.float32)
    for cc in range(8):
        acc += jnp.dot(hv[cc], w_ref[cc], preferred_element_type=jnp.float32)
    h_ref[0] = acc[:, :64]
    h_ref[1] = acc[:, 64:]

    @pl.when(j == 0)
    def _():
        al_ref[...] = jnp.zeros_like(al_ref)

    al_ref[...] += jnp.dot(acc, a_ref[...], preferred_element_type=jnp.float32,
                           precision=lax.Precision.HIGHEST)


def _run_proj2(o1, b1r, W2r, A2r, RB):
    NP = o1.shape[1]
    C = W2r.shape[2] // 128
    grid = (NP // RB, C)
    return pl.pallas_call(
        _proj2_kernel,
        grid=grid,
        in_specs=[
            pl.BlockSpec((8, RB, 64), lambda i, j: (0, i, 0)),
            pl.BlockSpec((8, 64), lambda i, j: (0, 0)),
            pl.BlockSpec((8, 64, 128), lambda i, j: (0, 0, j)),
            pl.BlockSpec((128, 128), lambda i, j: (j, 0)),
        ],
        out_specs=[
            pl.BlockSpec((2, RB, 64), lambda i, j: (j, i, 0)),
            pl.BlockSpec((RB, 128), lambda i, j: (i, 0)),
        ],
        out_shape=[
            jax.ShapeDtypeStruct((2 * C, NP, 64), jnp.float32),
            jax.ShapeDtypeStruct((NP, 128), jnp.float32),
        ],
    )(o1, b1r, W2r, A2r)


def _pool_kernel(o_ref, b_ref, bat_ref, f1w_ref, f1b_ref, f2w_ref, f2b_ref,
                 out_ref, acc_ref, cnt_ref):
    i = pl.program_id(0)
    nblocks = pl.num_programs(0)

    @pl.when(i == 0)
    def _():
        acc_ref[...] = jnp.zeros_like(acc_ref)
        cnt_ref[...] = jnp.zeros_like(cnt_ref)

    bvec = bat_ref[0, 0, :]
    validc = bvec[:, None] < 32
    oh = (bvec[:, None] == lax.broadcasted_iota(jnp.int32, (bvec.shape[0], 32), 1))
    oh = oh.astype(jnp.float32)
    for cc in range(8):
        hv = jnp.maximum(o_ref[cc] + b_ref[...][cc][None, :], 0.0)
        hv = jnp.where(validc, hv, 0.0)
        acc_ref[:, cc * 64:(cc + 1) * 64] += lax.dot_general(
            oh, hv, (((0,), (0,)), ((), ())),
            preferred_element_type=jnp.float32)
    cnt_ref[0, :32] += jnp.sum(oh, axis=0)

    @pl.when(i == nblocks - 1)
    def _():
        cnt = jnp.maximum(cnt_ref[0, :32], 1.0)
        pooled = acc_ref[...] / cnt[:, None]
        g = jnp.maximum(
            jnp.dot(pooled, f1w_ref[...], preferred_element_type=jnp.float32) + f1b_ref[...], 0.0)
        out_ref[...] = jnp.dot(
            g, f2w_ref[...], preferred_element_type=jnp.float32) + f2b_ref[...]


def _run_pool(o2, b2r, bat3, fc1_w, fc1_b, fc2_w, fc2_b, RB):
    NP = o2.shape[1]
    grid = (NP // RB,)
    return pl.pallas_call(
        _pool_kernel,
        grid=grid,
        in_specs=[
            pl.BlockSpec((8, RB, 64), lambda i: (0, i, 0)),
            pl.BlockSpec((8, 64), lambda i: (0, 0)),
            pl.BlockSpec((1, 1, RB), lambda i: (i, 0, 0)),
            pl.BlockSpec(fc1_w.shape, lambda i: (0, 0)),
            pl.BlockSpec((1, 1024), lambda i: (0, 0)),
            pl.BlockSpec(fc2_w.shape, lambda i: (0, 0)),
            pl.BlockSpec((1, 128), lambda i: (0, 0)),
        ],
        out_specs=pl.BlockSpec((32, 128), lambda i: (0, 0)),
        out_shape=jax.ShapeDtypeStruct((32, 128), jnp.float32),
        scratch_shapes=[
            pltpu.VMEM((32, 512), jnp.float32),
            pltpu.VMEM((8, 128), jnp.float32),
        ],
    )(o2, b2r, bat3, fc1_w, fc1_b, fc2_w, fc2_b)


def _make_sc_layer(H, NP, EP):
    NBT = EP // 128 // 16
    NBW = NBT
    STRIP = 8
    RSL = 2
    RPT = NP // 16
    DEN = NP * H
    mesh = plsc.VectorSubcoreMesh(core_axis_name="c", subcore_axis_name="s")

    def body(src3_hbm, dst3_hbm, asrc_hbm, adst_hbm, h_hbm, out_hbm,
             src3_v, dst3_v, asrc_v, adst_v, den_v, alpha_v,
             vals_v, idx_v, rbuf, zbuf, zed, acc_sh, den_sh, semg, sema):
        c = lax.axis_index("c")
        s = lax.axis_index("s")

        pltpu.sync_copy(src3_hbm.at[pl.ds(s * NBT, NBT)], src3_v)
        pltpu.sync_copy(dst3_hbm.at[pl.ds(s * NBT, NBT)], dst3_v)

        def _z16(i, _):
            zed[pl.ds(i * 16, 16)] = jnp.zeros((16,), jnp.float32)
            return 0
        lax.fori_loop(0, zed.shape[0] // 16, _z16, 0)

        def _zb(i, _):
            for k in range(zbuf.shape[1] // 16):
                zbuf[i, pl.ds(k * 16, 16)] = jnp.zeros((16,), jnp.float32)
            return 0
        lax.fori_loop(0, zbuf.shape[0], _zb, 0)

        seg = DEN // 16
        pltpu.sync_copy(zed.at[pl.ds(0, seg)], den_sh.at[pl.ds(s * seg, seg)])
        plsc.subcore_barrier()

        for hh in range(H):
            pltpu.sync_copy(asrc_hbm.at[hh], asrc_v)
            pltpu.sync_copy(adst_hbm.at[hh], adst_v)

            def strip_body(t, _):
                descs = []
                for b in range(STRIP):
                    bat = t * STRIP + b
                    for u in range(8):
                        sv = src3_v[bat, 0, pl.ds(u * 16, 16)]
                        dv = dst3_v[bat, 0, pl.ds(u * 16, 16)]
                        a_s = plsc.load_gather(asrc_v, [sv])
                        a_d = plsc.load_gather(adst_v, [dv])
                        e = a_s + a_d
                        e = jnp.where(e >= 0.0, e, 0.2 * e)
                        ex = jnp.exp(e)
                        vals_v[b, 0, pl.ds(u * 16, 16)] = ex
                        idx_v[b, 0, pl.ds(u * 16, 16)] = dv + hh * NP
                for b in range(STRIP):
                    descs.append(pltpu.async_copy(
                        vals_v.at[b, 0], den_sh.at[idx_v.at[b, 0]], sema,
                        add=True))
                for d in descs:
                    d.wait()
                return 0
            lax.fori_loop(0, NBT // STRIP, strip_body, 0)
        plsc.subcore_barrier()

        hd = c if H == 2 else c * 0
        if H == 2:
            pltpu.sync_copy(asrc_hbm.at[hd], asrc_v)
            pltpu.sync_copy(adst_hbm.at[hd], adst_v)
        pltpu.sync_copy(den_sh.at[pl.ds(hd * NP, NP)], den_v)

        def alpha_body(st, _):
            bi = st // 8
            lo = (st % 8) * 16
            sv = src3_v[bi, 0, pl.ds(lo, 16)]
            dv = dst3_v[bi, 0, pl.ds(lo, 16)]
            a_s = plsc.load_gather(asrc_v, [sv])
            a_d = plsc.load_gather(adst_v, [dv])
            e = a_s + a_d
            e = jnp.where(e >= 0.0, e, 0.2 * e)
            ex = jnp.exp(e)
            den = plsc.load_gather(den_v, [dv])
            alpha_v[pl.ds(st * 16, 16)] = ex / jnp.maximum(den, 1e-16)
            return 0
        lax.fori_loop(0, NBW * 8, alpha_body, 0)

        for half in range(4):
            q = 4 * c + half
            for k in range(RPT // 64):
                pltpu.sync_copy(zbuf, acc_sh.at[pl.ds(s * RPT + k * 64, 64)])
            plsc.subcore_barrier()

            def msg_body(b, _):
                @pl.when(b > 0)
                def _():
                    pb = b - 1
                    sl = lax.rem(pb, RSL)
                    pltpu.make_async_copy(
                        h_hbm.at[q].at[src3_v.at[pb, 0]],
                        rbuf.at[sl], semg).wait()

                    def scale_grp(g, _):
                        av16 = alpha_v[pl.ds(pb * 128 + g * 16, 16)]
                        for j in range(16):
                            av = av16[j]
                            for k in range(4):
                                rbuf[sl, g * 16 + j, pl.ds(k * 16, 16)] = (
                                    rbuf[sl, g * 16 + j,
                                         pl.ds(k * 16, 16)] * av)
                        return 0
                    lax.fori_loop(0, 8, scale_grp, 0)
                    pltpu.async_copy(
                        rbuf.at[sl], acc_sh.at[dst3_v.at[pb, 0]], sema,
                        add=True)

                @pl.when(b < NBW)
                def _():
                    @pl.when(b >= RSL)
                    def _():
                        ob = b - RSL
                        pltpu.make_async_copy(
                            rbuf.at[lax.rem(ob, RSL)],
                            acc_sh.at[dst3_v.at[ob, 0]], sema).wait()
                    pltpu.async_copy(
                        h_hbm.at[q].at[src3_v.at[b, 0]],
                        rbuf.at[lax.rem(b, RSL)], semg)
                return 0
            lax.fori_loop(0, NBW + 1, msg_body, 0)
            for t in range(RSL):
                ob = NBW - RSL + t
                pltpu.make_async_copy(
                    rbuf.at[ob % RSL], acc_sh.at[dst3_v.at[ob, 0]],
                    sema).wait()
            plsc.subcore_barrier()

            for k in range(RPT // 128):
                r0 = s * RPT + k * 128
                pltpu.sync_copy(acc_sh.at[pl.ds(r0, 128)], rbuf.at[0])
                pltpu.sync_copy(rbuf.at[0], out_hbm.at[q].at[pl.ds(r0, 128)])
            plsc.subcore_barrier()

    kern = pl.kernel(
        body,
        out_type=jax.ShapeDtypeStruct((8, NP, 64), jnp.float32),
        mesh=mesh,
        compiler_params=pltpu.CompilerParams(needs_layout_passes=False,
                                             use_tc_tiling_on_sc=False),
        scratch_types=[
            pltpu.VMEM((NBT, 1, 128), jnp.int32),
            pltpu.VMEM((NBT, 1, 128), jnp.int32),
            pltpu.VMEM((NP,), jnp.float32),
            pltpu.VMEM((NP,), jnp.float32),
            pltpu.VMEM((NP,), jnp.float32),
            pltpu.VMEM((NBW * 128,), jnp.float32),
            pltpu.VMEM((STRIP, 1, 128), jnp.float32),
            pltpu.VMEM((STRIP, 1, 128), jnp.int32),
            pltpu.VMEM((RSL, 128, 64), jnp.float32),
            pltpu.VMEM((64, 64), jnp.float32),
            pltpu.VMEM((DEN // 16,), jnp.float32),
            pltpu.VMEM_SHARED((NP, 64), jnp.float32),
            pltpu.VMEM_SHARED((DEN,), jnp.float32),
            pltpu.SemaphoreType.DMA,
            pltpu.SemaphoreType.DMA,
        ],
    )
    return kern


def kernel(x, edge_index, batch, W1, a_src1, a_dst1, b1,
           W2, a_src2, a_dst2, b2, fc1_w, fc1_b, fc2_w, fc2_b):
    N, F_in = x.shape
    E = edge_index.shape[1]
    H1, D1 = a_src1.shape
    D2 = W2.shape[1] // 2
    G = 32
    RB = 512

    NP = ((N + 8 + 2047) // 2048) * 2048
    EP = ((E + 4095) // 4096) * 4096

    f32 = jnp.float32
    src = edge_index[0].astype(jnp.int32)
    dst = edge_index[1].astype(jnp.int32)
    pad = EP - E
    ii = jnp.arange(pad, dtype=jnp.int32)
    src_p = jnp.concatenate([src, (ii * 37) % N])
    dst_p = jnp.concatenate([dst, N + (ii % 8)])
    src3 = src_p.reshape(EP // 128, 1, 128)
    dst3 = dst_p.reshape(EP // 128, 1, 128)

    xp = jnp.pad(x.astype(f32), ((0, NP - N), (0, 0)))

    A1p = jnp.zeros((H1 * D1, 128), f32)
    for h in range(H1):
        A1p = A1p.at[h * D1:(h + 1) * D1, h].set(a_src1[h])
        A1p = A1p.at[h * D1:(h + 1) * D1, 2 + h].set(a_dst1[h])

    h1tab, al1 = _run_proj(xp, W1.astype(f32), A1p, RB)
    asrc1_t = al1[:, 0:2].T.reshape(2, NP)
    adst1_t = al1[:, 2:4].T.reshape(2, NP)

    sc1 = _make_sc_layer(2, NP, EP)
    o1tab = sc1(src3, dst3, asrc1_t, adst1_t, h1tab)

    b1r = b1.astype(f32).reshape(8, 64)
    W2r = W2.astype(f32).reshape(8, 64, W2.shape[1])
    A2p = jnp.zeros((2 * D2, 128), f32)
    A2p = A2p.at[:, 0].set(a_src2[0])
    A2p = A2p.at[:, 1].set(a_dst2[0])
    h2tab, al2 = _run_proj2(o1tab, b1r, W2r, A2p, RB)
    asrc2_t = al2[:, 0].reshape(1, NP)
    adst2_t = al2[:, 1].reshape(1, NP)

    sc2 = _make_sc_layer(1, NP, EP)
    o2tab = sc2(src3, dst3, asrc2_t, adst2_t, h2tab)

    b2r = b2.astype(f32).reshape(8, 64)
    batp = jnp.pad(batch.astype(jnp.int32), (0, NP - N), constant_values=G)
    bat3 = batp.reshape(NP // RB, 1, RB)

    out = _run_pool(o2tab, b2r, bat3,
                    fc1_w.astype(f32), fc1_b.astype(f32).reshape(1, 1024),
                    fc2_w.astype(f32), fc2_b.astype(f32).reshape(1, 128), RB)
    return out


# speedup vs baseline: 10.6338x; 1.0132x over previous
import functools

import jax
import jax.numpy as jnp
from jax import lax
from jax.experimental import pallas as pl
from jax.experimental.pallas import tpu as pltpu
from jax.experimental.pallas import tpu_sc as plsc


def _proj_kernel(x_ref, w_ref, a_ref, h_ref, al_ref):
    j = pl.program_id(1)
    hblk = jnp.dot(x_ref[...], w_ref[...], preferred_element_type=jnp.float32)
    h_ref[0] = hblk[:, :64]
    h_ref[1] = hblk[:, 64:]

    @pl.when(j == 0)
    def _():
        al_ref[...] = jnp.zeros_like(al_ref)

    al_ref[...] += jnp.dot(hblk, a_ref[...], preferred_element_type=jnp.float32,
                           precision=lax.Precision.HIGHEST)


def _run_proj(xp, W, Ap, RB):
    NP, K = xp.shape
    C = W.shape[1] // 128
    grid = (NP // RB, C)
    return pl.pallas_call(
        _proj_kernel,
        grid=grid,
        in_specs=[
            pl.BlockSpec((RB, K), lambda i, j: (i, 0)),
            pl.BlockSpec((K, 128), lambda i, j: (0, j)),
            pl.BlockSpec((128, 128), lambda i, j: (j, 0)),
        ],
        out_specs=[
            pl.BlockSpec((2, RB, 64), lambda i, j: (j, i, 0)),
            pl.BlockSpec((RB, 128), lambda i, j: (i, 0)),
        ],
        out_shape=[
            jax.ShapeDtypeStruct((2 * C, NP, 64), jnp.float32),
            jax.ShapeDtypeStruct((NP, 128), jnp.float32),
        ],
    )(xp, W, Ap)


def _proj2_kernel(o_ref, b_ref, w_ref, a_ref, h_ref, al_ref):
    j = pl.program_id(1)
    hv = jnp.maximum(o_ref[...] + b_ref[...][:, None, :], 0.0)
    acc = jnp.zeros((hv.shape[1], 128), jnp.float32)
    for cc in range(8):
        acc += jnp.dot(hv[cc], w_ref[cc], preferred_element_type=jnp.float32)
    h_ref[0] = acc[:, :64]
    h_ref[1] = acc[:, 64:]

    @pl.when(j == 0)
    def _():
        al_ref[...] = jnp.zeros_like(al_ref)

    al_ref[...] += jnp.dot(acc, a_ref[...], preferred_element_type=jnp.float32,
                           precision=lax.Precision.HIGHEST)


def _run_proj2(o1, b1r, W2r, A2r, RB):
    NP = o1.shape[1]
    C = W2r.shape[2] // 128
    grid = (NP // RB, C)
    return pl.pallas_call(
        _proj2_kernel,
        grid=grid,
        in_specs=[
            pl.BlockSpec((8, RB, 64), lambda i, j: (0, i, 0)),
            pl.BlockSpec((8, 64), lambda i, j: (0, 0)),
            pl.BlockSpec((8, 64, 128), lambda i, j: (0, 0, j)),
            pl.BlockSpec((128, 128), lambda i, j: (j, 0)),
        ],
        out_specs=[
            pl.BlockSpec((2, RB, 64), lambda i, j: (j, i, 0)),
            pl.BlockSpec((RB, 128), lambda i, j: (i, 0)),
        ],
        out_shape=[
            jax.ShapeDtypeStruct((2 * C, NP, 64), jnp.float32),
            jax.ShapeDtypeStruct((NP, 128), jnp.float32),
        ],
    )(o1, b1r, W2r, A2r)


def _pool_kernel(o_ref, b_ref, bat_ref, f1w_ref, f1b_ref, f2w_ref, f2b_ref,
                 out_ref, acc_ref, cnt_ref):
    i = pl.program_id(0)
    nblocks = pl.num_programs(0)

    @pl.when(i == 0)
    def _():
        acc_ref[...] = jnp.zeros_like(acc_ref)
        cnt_ref[...] = jnp.zeros_like(cnt_ref)

    bvec = bat_ref[0, 0, :]
    validc = bvec[:, None] < 32
    oh = (bvec[:, None] == lax.broadcasted_iota(jnp.int32, (bvec.shape[0], 32), 1))
    oh = oh.astype(jnp.float32)
    for cc in range(8):
        hv = jnp.maximum(o_ref[cc] + b_ref[...][cc][None, :], 0.0)
        hv = jnp.where(validc, hv, 0.0)
        acc_ref[:, cc * 64:(cc + 1) * 64] += lax.dot_general(
            oh, hv, (((0,), (0,)), ((), ())),
            preferred_element_type=jnp.float32)
    cnt_ref[0, :32] += jnp.sum(oh, axis=0)

    @pl.when(i == nblocks - 1)
    def _():
        cnt = jnp.maximum(cnt_ref[0, :32], 1.0)
        pooled = acc_ref[...] / cnt[:, None]
        g = jnp.maximum(
            jnp.dot(pooled, f1w_ref[...], preferred_element_type=jnp.float32) + f1b_ref[...], 0.0)
        out_ref[...] = jnp.dot(
            g, f2w_ref[...], preferred_element_type=jnp.float32) + f2b_ref[...]


def _run_pool(o2, b2r, bat3, fc1_w, fc1_b, fc2_w, fc2_b, RB):
    NP = o2.shape[1]
    grid = (NP // RB,)
    return pl.pallas_call(
        _pool_kernel,
        grid=grid,
        in_specs=[
            pl.BlockSpec((8, RB, 64), lambda i: (0, i, 0)),
            pl.BlockSpec((8, 64), lambda i: (0, 0)),
            pl.BlockSpec((1, 1, RB), lambda i: (i, 0, 0)),
            pl.BlockSpec(fc1_w.shape, lambda i: (0, 0)),
            pl.BlockSpec((1, 1024), lambda i: (0, 0)),
            pl.BlockSpec(fc2_w.shape, lambda i: (0, 0)),
            pl.BlockSpec((1, 128), lambda i: (0, 0)),
        ],
        out_specs=pl.BlockSpec((32, 128), lambda i: (0, 0)),
        out_shape=jax.ShapeDtypeStruct((32, 128), jnp.float32),
        scratch_shapes=[
            pltpu.VMEM((32, 512), jnp.float32),
            pltpu.VMEM((8, 128), jnp.float32),
        ],
    )(o2, b2r, bat3, fc1_w, fc1_b, fc2_w, fc2_b)


def _make_sc_layer(H, NP, EP):
    NBT = EP // 128 // 16
    NBW = NBT
    STRIP = 8
    RSL = 2
    RPT = NP // 16
    DEN = NP
    mesh = plsc.VectorSubcoreMesh(core_axis_name="c", subcore_axis_name="s")

    def body(src3_hbm, dst3_hbm, asrc_hbm, adst_hbm, h_hbm, out_hbm,
             src3_v, dst3_v, asrc_v, adst_v, den_v, alpha_v,
             rbuf, zbuf, zed, acc_sh, den_sh, semg, sema):
        c = lax.axis_index("c")
        s = lax.axis_index("s")
        hd = c if H == 2 else c * 0

        pltpu.sync_copy(src3_hbm.at[pl.ds(s * NBT, NBT)], src3_v)
        pltpu.sync_copy(dst3_hbm.at[pl.ds(s * NBT, NBT)], dst3_v)

        def _z16(i, _):
            zed[pl.ds(i * 16, 16)] = jnp.zeros((16,), jnp.float32)
            return 0
        lax.fori_loop(0, zed.shape[0] // 16, _z16, 0)

        def _zb(i, _):
            for k in range(zbuf.shape[1] // 16):
                zbuf[i, pl.ds(k * 16, 16)] = jnp.zeros((16,), jnp.float32)
            return 0
        lax.fori_loop(0, zbuf.shape[0], _zb, 0)

        seg = DEN // 16
        pltpu.sync_copy(zed.at[pl.ds(0, seg)], den_sh.at[pl.ds(s * seg, seg)])
        plsc.subcore_barrier()

        pltpu.sync_copy(asrc_hbm.at[hd], asrc_v)
        pltpu.sync_copy(adst_hbm.at[hd], adst_v)

        def strip_body(t, _):
            descs = []
            for b in range(STRIP):
                bat = t * STRIP + b
                for u in range(8):
                    sv = src3_v[bat, 0, pl.ds(u * 16, 16)]
                    dv = dst3_v[bat, 0, pl.ds(u * 16, 16)]
                    a_s = plsc.load_gather(asrc_v, [sv])
                    a_d = plsc.load_gather(adst_v, [dv])
                    e = a_s + a_d
                    e = jnp.where(e >= 0.0, e, 0.2 * e)
                    alpha_v[pl.ds(bat * 128 + u * 16, 16)] = jnp.exp(e)
            for b in range(STRIP):
                bat = t * STRIP + b
                descs.append(pltpu.async_copy(
                    alpha_v.at[pl.ds(bat * 128, 128)],
                    den_sh.at[dst3_v.at[bat, 0]], sema, add=True))
            for d in descs:
                d.wait()
            return 0
        lax.fori_loop(0, NBT // STRIP, strip_body, 0)
        plsc.subcore_barrier()

        pltpu.sync_copy(den_sh, den_v)

        def alpha_body(st, _):
            bi = st // 8
            lo = (st % 8) * 16
            dv = dst3_v[bi, 0, pl.ds(lo, 16)]
            den = plsc.load_gather(den_v, [dv])
            ex = alpha_v[pl.ds(st * 16, 16)]
            alpha_v[pl.ds(st * 16, 16)] = ex / jnp.maximum(den, 1e-16)
            return 0
        lax.fori_loop(0, NBW * 8, alpha_body, 0)

        for half in range(4):
            q = 4 * c + half
            for k in range(RPT // 64):
                pltpu.sync_copy(zbuf, acc_sh.at[pl.ds(s * RPT + k * 64, 64)])
            plsc.subcore_barrier()

            def msg_body(b, _):
                @pl.when(b > 0)
                def _():
                    pb = b - 1
                    sl = lax.rem(pb, RSL)
                    pltpu.make_async_copy(
                        h_hbm.at[q].at[src3_v.at[pb, 0]],
                        rbuf.at[sl], semg).wait()

                    def scale_grp(g, _):
                        av16 = alpha_v[pl.ds(pb * 128 + g * 16, 16)]
                        for j in range(16):
                            av = av16[j]
                            for k in range(4):
                                rbuf[sl, g * 16 + j, pl.ds(k * 16, 16)] = (
                                    rbuf[sl, g * 16 + j,
                                         pl.ds(k * 16, 16)] * av)
                        return 0
                    lax.fori_loop(0, 8, scale_grp, 0)
                    pltpu.async_copy(
                        rbuf.at[sl], acc_sh.at[dst3_v.at[pb, 0]], sema,
                        add=True)

                @pl.when(b < NBW)
                def _():
                    @pl.when(b >= RSL)
                    def _():
                        ob = b - RSL
                        pltpu.make_async_copy(
                            rbuf.at[lax.rem(ob, RSL)],
                            acc_sh.at[dst3_v.at[ob, 0]], sema).wait()
                    pltpu.async_copy(
                        h_hbm.at[q].at[src3_v.at[b, 0]],
                        rbuf.at[lax.rem(b, RSL)], semg)
                return 0
            lax.fori_loop(0, NBW + 1, msg_body, 0)
            for t in range(RSL):
                ob = NBW - RSL + t
                pltpu.make_async_copy(
                    rbuf.at[ob % RSL], acc_sh.at[dst3_v.at[ob, 0]],
                    sema).wait()
            plsc.subcore_barrier()

            for k in range(RPT // 128):
                r0 = s * RPT + k * 128
                pltpu.sync_copy(acc_sh.at[pl.ds(r0, 128)], rbuf.at[0])
                pltpu.sync_copy(rbuf.at[0], out_hbm.at[q].at[pl.ds(r0, 128)])
            plsc.subcore_barrier()

    kern = pl.kernel(
        body,
        out_type=jax.ShapeDtypeStruct((8, NP, 64), jnp.float32),
        mesh=mesh,
        compiler_params=pltpu.CompilerParams(needs_layout_passes=False,
                                             use_tc_tiling_on_sc=False),
        scratch_types=[
            pltpu.VMEM((NBT, 1, 128), jnp.int32),
            pltpu.VMEM((NBT, 1, 128), jnp.int32),
            pltpu.VMEM((NP,), jnp.float32),
            pltpu.VMEM((NP,), jnp.float32),
            pltpu.VMEM((NP,), jnp.float32),
            pltpu.VMEM((NBW * 128,), jnp.float32),
            pltpu.VMEM((RSL, 128, 64), jnp.float32),
            pltpu.VMEM((64, 64), jnp.float32),
            pltpu.VMEM((DEN // 16,), jnp.float32),
            pltpu.VMEM_SHARED((NP, 64), jnp.float32),
            pltpu.VMEM_SHARED((DEN,), jnp.float32),
            pltpu.SemaphoreType.DMA,
            pltpu.SemaphoreType.DMA,
        ],
    )
    return kern


def kernel(x, edge_index, batch, W1, a_src1, a_dst1, b1,
           W2, a_src2, a_dst2, b2, fc1_w, fc1_b, fc2_w, fc2_b):
    N, F_in = x.shape
    E = edge_index.shape[1]
    H1, D1 = a_src1.shape
    D2 = W2.shape[1] // 2
    G = 32
    RB = 512

    NP = ((N + 8 + 2047) // 2048) * 2048
    EP = ((E + 4095) // 4096) * 4096

    f32 = jnp.float32
    src = edge_index[0].astype(jnp.int32)
    dst = edge_index[1].astype(jnp.int32)
    pad = EP - E
    ii = jnp.arange(pad, dtype=jnp.int32)
    src_p = jnp.concatenate([src, (ii * 37) % N])
    dst_p = jnp.concatenate([dst, N + (ii % 8)])
    src3 = src_p.reshape(EP // 128, 1, 128)
    dst3 = dst_p.reshape(EP // 128, 1, 128)

    xp = jnp.pad(x.astype(f32), ((0, NP - N), (0, 0)))

    A1p = jnp.zeros((H1 * D1, 128), f32)
    for h in range(H1):
        A1p = A1p.at[h * D1:(h + 1) * D1, h].set(a_src1[h])
        A1p = A1p.at[h * D1:(h + 1) * D1, 2 + h].set(a_dst1[h])

    h1tab, al1 = _run_proj(xp, W1.astype(f32), A1p, RB)
    asrc1_t = al1[:, 0:2].T.reshape(2, NP)
    adst1_t = al1[:, 2:4].T.reshape(2, NP)

    sc1 = _make_sc_layer(2, NP, EP)
    o1tab = sc1(src3, dst3, asrc1_t, adst1_t, h1tab)

    b1r = b1.astype(f32).reshape(8, 64)
    W2r = W2.astype(f32).reshape(8, 64, W2.shape[1])
    A2p = jnp.zeros((2 * D2, 128), f32)
    A2p = A2p.at[:, 0].set(a_src2[0])
    A2p = A2p.at[:, 1].set(a_dst2[0])
    h2tab, al2 = _run_proj2(o1tab, b1r, W2r, A2p, RB)
    asrc2_t = al2[:, 0].reshape(1, NP)
    adst2_t = al2[:, 1].reshape(1, NP)

    sc2 = _make_sc_layer(1, NP, EP)
    o2tab = sc2(src3, dst3, asrc2_t, adst2_t, h2tab)

    b2r = b2.astype(f32).reshape(8, 64)
    batp = jnp.pad(batch.astype(jnp.int32), (0, NP - N), constant_values=G)
    bat3 = batp.reshape(NP // RB, 1, RB)

    out = _run_pool(o2tab, b2r, bat3,
                    fc1_w.astype(f32), fc1_b.astype(f32).reshape(1, 1024),
                    fc2_w.astype(f32), fc2_b.astype(f32).reshape(1, 128), RB)
    return out
